# baseline (speedup 1.0000x reference)
import jax
import jax.numpy as jnp
from jax import lax
from jax.experimental import pallas as pl
from jax.experimental.pallas import tpu as pltpu
from jax.experimental.pallas import tpu_sc as plsc

S = 8192
B = 4
C = 768
L2 = S // 2
NC = 2
NS = 16
LANES = 16

_SBLK = 256

_MSB_INT = -2147483648


def _scores_body(x_ref, w_ref, o_ref, o2_ref):
    xr = x_ref[...].reshape(_SBLK * B, C)
    o2_ref[...] = xr
    o_ref[...] = lax.dot_general(
        xr, w_ref[...], (((1,), (0,)), ((), ())),
        precision=lax.Precision.DEFAULT,
        preferred_element_type=jnp.float32)


def _scores_tc(x, W):
    wpad = jnp.zeros((C, 128), jnp.float32).at[:, 0].set(W)
    out, x2d = pl.pallas_call(
        _scores_body,
        grid=(S // _SBLK,),
        in_specs=[
            pl.BlockSpec((_SBLK, B, C), lambda i: (i, 0, 0)),
            pl.BlockSpec((C, 128), lambda i: (0, 0)),
        ],
        out_specs=[
            pl.BlockSpec((_SBLK * B, 128), lambda i: (i, 0)),
            pl.BlockSpec((_SBLK * B, C), lambda i: (i, 0)),
        ],
        out_shape=[
            jax.ShapeDtypeStruct((S * B, 128), jnp.float32),
            jax.ShapeDtypeStruct((S * B, C), jnp.float32),
        ],
    )(x, wpad)
    return out[:, 0].reshape(S, B), x2d


def _desc_key(fbits):
    msb = jnp.int32(_MSB_INT)
    m = lax.shift_right_arithmetic(fbits, 31)
    asc = lax.bitwise_xor(fbits, lax.bitwise_or(m, msb))
    return lax.bitwise_not(asc)


def _inv_key(k):
    msb = jnp.int32(_MSB_INT)
    asc = lax.bitwise_not(k)
    bits = jnp.where(asc < 0, lax.bitwise_xor(asc, msb), lax.bitwise_not(asc))
    return lax.bitcast_convert_type(bits, jnp.float32)


def _sc_body(scores_ref, x2d_ref, idx_out, w_out, xds_out,
             kf, ki, vi, ki2, vi2, hist, histb, posr, idxv, wv, idxbuf,
             gidx0, gidx1, oidx0, oidx1, rows0, rows1,
             semg0, semg1, sems0, sems1):
    c = lax.axis_index("c")
    s = lax.axis_index("s")
    iota = lax.iota(jnp.int32, LANES)
    ones = jnp.ones((LANES,), jnp.int32)
    zeros = jnp.zeros((LANES,), jnp.int32)
    nvec = S // LANES
    colbase = iota * nvec

    @pl.when(s < 2)
    def _sort_phase():
        b = 2 * s + c
        pltpu.sync_copy(scores_ref.at[b], kf)

        @plsc.parallel_loop(0, nvec, step=1, unroll=8)
        def xf(i):
            fb = lax.bitcast_convert_type(
                kf[pl.ds(i * LANES, LANES)], jnp.int32)
            ki[pl.ds(i * LANES, LANES)] = _desc_key(fb)
            vi[pl.ds(i * LANES, LANES)] = iota + i * LANES

        def radix_pass(sk, sv, dk, dv, shift):
            hvec = nvec // 2
            @plsc.parallel_loop(0, 256, step=1, unroll=8)
            def z(i):
                hist[pl.ds(i * LANES, LANES)] = zeros
                histb[pl.ds(i * LANES, LANES)] = zeros

            @plsc.parallel_loop(0, hvec, step=1, unroll=8)
            def h(v):
                kv0 = plsc.load_gather(sk, [colbase + v])
                d0 = lax.bitwise_and(
                    lax.shift_right_logical(kv0, shift), jnp.int32(255))
                plsc.addupdate_scatter(hist, [d0 * LANES + iota], ones)
                kv1 = plsc.load_gather(sk, [colbase + hvec + v])
                d1 = lax.bitwise_and(
                    lax.shift_right_logical(kv1, shift), jnp.int32(255))
                plsc.addupdate_scatter(histb, [d1 * LANES + iota], ones)

            def sc_(i, carry):
                h0 = hist[pl.ds(i * LANES, LANES)]
                h1 = histb[pl.ds(i * LANES, LANES)]
                t = h0 + h1
                cs = plsc.cumsum(t)
                excl = cs - t + carry
                hist[pl.ds(i * LANES, LANES)] = excl
                histb[pl.ds(i * LANES, LANES)] = excl + h0
                return carry + jnp.sum(t)
            lax.fori_loop(0, 256, sc_, jnp.int32(0), unroll=4)

            def pma(v, _):
                kv0 = plsc.load_gather(sk, [colbase + v])
                d0 = lax.bitwise_and(
                    lax.shift_right_logical(kv0, shift), jnp.int32(255))
                hidx0 = d0 * LANES + iota
                pos0 = plsc.load_gather(hist, [hidx0])
                plsc.store_scatter(hist, [hidx0], pos0 + 1)
                plsc.store_scatter(posr, [colbase + v], pos0)
                kv1 = plsc.load_gather(sk, [colbase + hvec + v])
                d1 = lax.bitwise_and(
                    lax.shift_right_logical(kv1, shift), jnp.int32(255))
                hidx1 = d1 * LANES + iota
                pos1 = plsc.load_gather(histb, [hidx1])
                plsc.store_scatter(histb, [hidx1], pos1 + 1)
                plsc.store_scatter(posr, [colbase + hvec + v], pos1)
                return 0
            lax.fori_loop(0, hvec, pma, 0, unroll=4)

            @plsc.parallel_loop(0, nvec, step=1, unroll=8)
            def pmb(v):
                kv = sk[pl.ds(v * LANES, LANES)]
                vv = sv[pl.ds(v * LANES, LANES)]
                pos = posr[pl.ds(v * LANES, LANES)]
                plsc.store_scatter(dk, [pos], kv)
                plsc.store_scatter(dv, [pos], vv)

        radix_pass(ki, vi, ki2, vi2, 0)
        radix_pass(ki2, vi2, ki, vi, 8)
        radix_pass(ki, vi, ki2, vi2, 16)
        radix_pass(ki2, vi2, ki, vi, 24)

        @plsc.parallel_loop(0, nvec, step=1, unroll=8)
        def z2(i):
            posr[pl.ds(i * LANES, LANES)] = zeros

        @plsc.parallel_loop(0, L2 // LANES, step=1, unroll=8)
        def mk(r):
            origv = vi[pl.ds(r * LANES, LANES)]
            plsc.store_scatter(posr, [origv], ones)

        def ps(i, carry):
            h16 = posr[pl.ds(i * LANES, LANES)]
            cs = plsc.cumsum(h16)
            posr[pl.ds(i * LANES, LANES)] = cs - h16 + carry
            return carry + jnp.sum(h16)
        lax.fori_loop(0, nvec, ps, jnp.int32(0), unroll=4)

        @plsc.parallel_loop(0, L2 // LANES, step=1, unroll=8)
        def wi(r):
            a = _inv_key(ki[pl.ds(r * LANES, LANES)])
            bb = _inv_key(ki[pl.ds(r * LANES + L2, LANES)])
            w = jnp.clip(a, 0.0, 1.0) - jnp.clip(bb, 0.0, 1.0)
            origv = vi[pl.ds(r * LANES, LANES)]
            posv = plsc.load_gather(posr, [origv])
            plsc.store_scatter(wv, [posv], w)
            plsc.store_scatter(idxv, [posv], origv)

        pltpu.sync_copy(idxv, idx_out.at[b])
        pltpu.sync_copy(wv, w_out.at[b])

    plsc.subcore_barrier()

    ppw = L2 // NS
    p_base = s * ppw
    pltpu.sync_copy(idx_out.at[c, pl.ds(p_base, ppw)],
                    idxbuf.at[pl.ds(0, ppw)])
    pltpu.sync_copy(idx_out.at[c + 2, pl.ds(p_base, ppw)],
                    idxbuf.at[pl.ds(ppw, ppw)])

    bufs = ((gidx0, oidx0, rows0, semg0, sems0),
            (gidx1, oidx1, rows1, semg1, sems1))
    nchunk = ppw // LANES
    pend = [None, None]
    for ci in range(nchunk):
        gref, oref, rows, semg, sems_ = bufs[ci % 2]
        if pend[ci % 2] is not None:
            pend[ci % 2].wait()
        off = ci * LANES
        i0 = plsc.load_gather(idxbuf, [off + iota])
        i1 = plsc.load_gather(idxbuf, [ppw + off + iota])
        gref[pl.ds(0, LANES)] = i0 * B + c
        gref[pl.ds(LANES, LANES)] = i1 * B + c + 2
        pv = (p_base + off + iota) * B
        oref[pl.ds(0, LANES)] = pv + c
        oref[pl.ds(LANES, LANES)] = pv + c + 2
        pltpu.async_copy(x2d_ref.at[gref], rows, semg).wait()
        pend[ci % 2] = pltpu.async_copy(rows, xds_out.at[oref], sems_)
    if pend[0] is not None:
        pend[0].wait()
    if pend[1] is not None:
        pend[1].wait()


_sc_kernel = pl.kernel(
    _sc_body,
    out_type=[
        jax.ShapeDtypeStruct((B, L2), jnp.int32),
        jax.ShapeDtypeStruct((B, L2), jnp.float32),
        jax.ShapeDtypeStruct((L2 * B, C), jnp.float32),
    ],
    mesh=plsc.VectorSubcoreMesh(
        core_axis_name="c", subcore_axis_name="s",
        num_cores=NC, num_subcores=NS),
    compiler_params=pltpu.CompilerParams(needs_layout_passes=False),
    scratch_types=[
        pltpu.VMEM((S,), jnp.float32),
        pltpu.VMEM((S,), jnp.int32),
        pltpu.VMEM((S,), jnp.int32),
        pltpu.VMEM((S,), jnp.int32),
        pltpu.VMEM((S,), jnp.int32),
        pltpu.VMEM((256 * LANES,), jnp.int32),
        pltpu.VMEM((256 * LANES,), jnp.int32),
        pltpu.VMEM((S,), jnp.int32),
        pltpu.VMEM((L2,), jnp.int32),
        pltpu.VMEM((L2,), jnp.float32),
        pltpu.VMEM((2 * (L2 // NS),), jnp.int32),
        pltpu.VMEM((2 * LANES,), jnp.int32),
        pltpu.VMEM((2 * LANES,), jnp.int32),
        pltpu.VMEM((2 * LANES,), jnp.int32),
        pltpu.VMEM((2 * LANES,), jnp.int32),
        pltpu.VMEM((2 * LANES, C), jnp.float32),
        pltpu.VMEM((2 * LANES, C), jnp.float32),
        pltpu.SemaphoreType.DMA,
        pltpu.SemaphoreType.DMA,
        pltpu.SemaphoreType.DMA,
        pltpu.SemaphoreType.DMA,
    ],
)


def kernel(x, W):
    scores, x2d = _scores_tc(x, W)
    scores_t = scores.T
    idxs, wts, xds2 = _sc_kernel(scores_t, x2d)
    return idxs, wts, xds2.reshape(L2, B, C)


# speedup vs baseline: 1.0327x; 1.0327x over previous
import jax
import jax.numpy as jnp
from jax import lax
from jax.experimental import pallas as pl
from jax.experimental.pallas import tpu as pltpu
from jax.experimental.pallas import tpu_sc as plsc

S = 8192
B = 4
C = 768
L2 = S // 2
NC = 2
NS = 16
LANES = 16

_SBLK = 256

_MSB_INT = -2147483648


def _scores_body(x_ref, w_ref, o_ref, o2_ref):
    xr = x_ref[...].reshape(_SBLK * B, C)
    o2_ref[...] = xr
    o_ref[...] = lax.dot_general(
        xr, w_ref[...], (((1,), (0,)), ((), ())),
        precision=lax.Precision.DEFAULT,
        preferred_element_type=jnp.float32)


def _scores_tc(x, W):
    wpad = jnp.zeros((C, 128), jnp.float32).at[:, 0].set(W)
    out, x2d = pl.pallas_call(
        _scores_body,
        grid=(S // _SBLK,),
        in_specs=[
            pl.BlockSpec((_SBLK, B, C), lambda i: (i, 0, 0)),
            pl.BlockSpec((C, 128), lambda i: (0, 0)),
        ],
        out_specs=[
            pl.BlockSpec((_SBLK * B, 128), lambda i: (i, 0)),
            pl.BlockSpec((_SBLK * B, C), lambda i: (i, 0)),
        ],
        out_shape=[
            jax.ShapeDtypeStruct((S * B, 128), jnp.float32),
            jax.ShapeDtypeStruct((S * B, C), jnp.float32),
        ],
    )(x, wpad)
    return out[:, 0].reshape(S, B), x2d


def _desc_key(fbits):
    msb = jnp.int32(_MSB_INT)
    m = lax.shift_right_arithmetic(fbits, 31)
    asc = lax.bitwise_xor(fbits, lax.bitwise_or(m, msb))
    return lax.bitwise_not(asc)


def _inv_key(k):
    msb = jnp.int32(_MSB_INT)
    asc = lax.bitwise_not(k)
    bits = jnp.where(asc < 0, lax.bitwise_xor(asc, msb), lax.bitwise_not(asc))
    return lax.bitcast_convert_type(bits, jnp.float32)


def _sc_body(scores_ref, x2d_ref, idx_out, w_out, xds_out,
             kf, ki, vi, ki2, vi2, hist, posr, idxv, wv, idxbuf,
             gidx0, gidx1, oidx0, oidx1, rows0, rows1,
             semg0, semg1, sems0, sems1):
    c = lax.axis_index("c")
    s = lax.axis_index("s")
    iota = lax.iota(jnp.int32, LANES)
    ones = jnp.ones((LANES,), jnp.int32)
    zeros = jnp.zeros((LANES,), jnp.int32)
    nvec = S // LANES
    colbase = iota * nvec

    @pl.when(s < 2)
    def _sort_phase():
        b = 2 * s + c
        pltpu.sync_copy(scores_ref.at[b], kf)

        @plsc.parallel_loop(0, nvec, step=1, unroll=8)
        def xf(i):
            fb = lax.bitcast_convert_type(
                kf[pl.ds(i * LANES, LANES)], jnp.int32)
            ki[pl.ds(i * LANES, LANES)] = _desc_key(fb)
            vi[pl.ds(i * LANES, LANES)] = iota + i * LANES

        def radix_pass(sk, sv, dk, dv, shift):
            @plsc.parallel_loop(0, 256, step=1, unroll=8)
            def z(i):
                hist[pl.ds(i * LANES, LANES)] = zeros

            @plsc.parallel_loop(0, nvec, step=1, unroll=8)
            def h(v):
                kv = plsc.load_gather(sk, [colbase + v])
                d = lax.bitwise_and(
                    lax.shift_right_logical(kv, shift), jnp.int32(255))
                plsc.addupdate_scatter(hist, [d * LANES + iota], ones)

            def sc_(i, carry):
                h16 = hist[pl.ds(i * LANES, LANES)]
                cs = plsc.cumsum(h16)
                hist[pl.ds(i * LANES, LANES)] = cs - h16 + carry
                return carry + jnp.sum(h16)
            lax.fori_loop(0, 256, sc_, jnp.int32(0), unroll=4)

            def pma(v, _):
                kv = plsc.load_gather(sk, [colbase + v])
                d = lax.bitwise_and(
                    lax.shift_right_logical(kv, shift), jnp.int32(255))
                hidx = d * LANES + iota
                pos = plsc.load_gather(hist, [hidx])
                plsc.store_scatter(hist, [hidx], pos + 1)
                plsc.store_scatter(posr, [colbase + v], pos)
                return 0
            lax.fori_loop(0, nvec, pma, 0, unroll=8)

            @plsc.parallel_loop(0, nvec, step=1, unroll=8)
            def pmb(v):
                kv = sk[pl.ds(v * LANES, LANES)]
                vv = sv[pl.ds(v * LANES, LANES)]
                pos = posr[pl.ds(v * LANES, LANES)]
                plsc.store_scatter(dk, [pos], kv)
                plsc.store_scatter(dv, [pos], vv)

        radix_pass(ki, vi, ki2, vi2, 0)
        radix_pass(ki2, vi2, ki, vi, 8)
        radix_pass(ki, vi, ki2, vi2, 16)
        radix_pass(ki2, vi2, ki, vi, 24)

        @plsc.parallel_loop(0, nvec, step=1, unroll=8)
        def z2(i):
            posr[pl.ds(i * LANES, LANES)] = zeros

        @plsc.parallel_loop(0, L2 // LANES, step=1, unroll=8)
        def mk(r):
            origv = vi[pl.ds(r * LANES, LANES)]
            plsc.store_scatter(posr, [origv], ones)

        def ps(i, carry):
            h16 = posr[pl.ds(i * LANES, LANES)]
            cs = plsc.cumsum(h16)
            posr[pl.ds(i * LANES, LANES)] = cs - h16 + carry
            return carry + jnp.sum(h16)
        lax.fori_loop(0, nvec, ps, jnp.int32(0), unroll=4)

        @plsc.parallel_loop(0, L2 // LANES, step=1, unroll=8)
        def wi(r):
            a = _inv_key(ki[pl.ds(r * LANES, LANES)])
            bb = _inv_key(ki[pl.ds(r * LANES + L2, LANES)])
            w = jnp.clip(a, 0.0, 1.0) - jnp.clip(bb, 0.0, 1.0)
            origv = vi[pl.ds(r * LANES, LANES)]
            posv = plsc.load_gather(posr, [origv])
            plsc.store_scatter(wv, [posv], w)
            plsc.store_scatter(idxv, [posv], origv)

        pltpu.sync_copy(idxv, idx_out.at[b])
        pltpu.sync_copy(wv, w_out.at[b])

    plsc.subcore_barrier()

    ppw = L2 // NS
    p_base = s * ppw
    pltpu.sync_copy(idx_out.at[c, pl.ds(p_base, ppw)],
                    idxbuf.at[pl.ds(0, ppw)])
    pltpu.sync_copy(idx_out.at[c + 2, pl.ds(p_base, ppw)],
                    idxbuf.at[pl.ds(ppw, ppw)])

    bufs = ((gidx0, oidx0, rows0, semg0, sems0),
            (gidx1, oidx1, rows1, semg1, sems1))
    nchunk = ppw // LANES
    pend = [None, None]
    for ci in range(nchunk):
        gref, oref, rows, semg, sems_ = bufs[ci % 2]
        if pend[ci % 2] is not None:
            pend[ci % 2].wait()
        off = ci * LANES
        i0 = plsc.load_gather(idxbuf, [off + iota])
        i1 = plsc.load_gather(idxbuf, [ppw + off + iota])
        gref[pl.ds(0, LANES)] = i0 * B + c
        gref[pl.ds(LANES, LANES)] = i1 * B + c + 2
        pv = (p_base + off + iota) * B
        oref[pl.ds(0, LANES)] = pv + c
        oref[pl.ds(LANES, LANES)] = pv + c + 2
        pltpu.async_copy(x2d_ref.at[gref], rows, semg).wait()
        pend[ci % 2] = pltpu.async_copy(rows, xds_out.at[oref], sems_)
    if pend[0] is not None:
        pend[0].wait()
    if pend[1] is not None:
        pend[1].wait()


_sc_kernel = pl.kernel(
    _sc_body,
    out_type=[
        jax.ShapeDtypeStruct((B, L2), jnp.int32),
        jax.ShapeDtypeStruct((B, L2), jnp.float32),
        jax.ShapeDtypeStruct((L2 * B, C), jnp.float32),
    ],
    mesh=plsc.VectorSubcoreMesh(
        core_axis_name="c", subcore_axis_name="s",
        num_cores=NC, num_subcores=NS),
    compiler_params=pltpu.CompilerParams(needs_layout_passes=False),
    scratch_types=[
        pltpu.VMEM((S,), jnp.float32),
        pltpu.VMEM((S,), jnp.int32),
        pltpu.VMEM((S,), jnp.int32),
        pltpu.VMEM((S,), jnp.int32),
        pltpu.VMEM((S,), jnp.int32),
        pltpu.VMEM((256 * LANES,), jnp.int32),
        pltpu.VMEM((S,), jnp.int32),
        pltpu.VMEM((L2,), jnp.int32),
        pltpu.VMEM((L2,), jnp.float32),
        pltpu.VMEM((2 * (L2 // NS),), jnp.int32),
        pltpu.VMEM((2 * LANES,), jnp.int32),
        pltpu.VMEM((2 * LANES,), jnp.int32),
        pltpu.VMEM((2 * LANES,), jnp.int32),
        pltpu.VMEM((2 * LANES,), jnp.int32),
        pltpu.VMEM((2 * LANES, C), jnp.float32),
        pltpu.VMEM((2 * LANES, C), jnp.float32),
        pltpu.SemaphoreType.DMA,
        pltpu.SemaphoreType.DMA,
        pltpu.SemaphoreType.DMA,
        pltpu.SemaphoreType.DMA,
    ],
)


def kernel(x, W):
    scores, x2d = _scores_tc(x, W)
    scores_t = scores.T
    idxs, wts, xds2 = _sc_kernel(scores_t, x2d)
    return idxs, wts, xds2.reshape(L2, B, C)
